# baseline (speedup 1.0000x reference)
import jax
import jax.numpy as jnp
from jax import lax
from jax.experimental import pallas as pl
from jax.experimental.pallas import tpu as pltpu
from jax.experimental.pallas import tpu_sc as plsc

VOCAB = 1000
B = 4096
T = 20
NC = 2
NS = 16
NW = NC * NS
BPW = B // NW
L = 16
NREG = VOCAB // L


STAGE_ROWS = VOCAB // NS
STAGE_REM = VOCAB - STAGE_ROWS * NS


def _gather_body(table_hbm, idx_hbm, out_hbm, idx_v, buf3a, buf3b, wba, wbb,
                 g0, g1, w0, w1):
    buf3 = (buf3a, buf3b)
    wbuf = (wba, wbb)
    gsem = (g0, g1)
    wsem = (w0, w1)

    sid = lax.axis_index("s")
    wid = sid * NC + lax.axis_index("c")
    base = wid * BPW
    pltpu.sync_copy(idx_hbm.at[pl.ds(base, BPW)], idx_v)

    def gather(j, b):
        return pltpu.make_async_copy(
            table_hbm.at[idx_v.at[j]], buf3[b], gsem[b]
        )

    def write(j, b):
        return pltpu.make_async_copy(wbuf[b], out_hbm.at[base + j], wsem[b])

    def repack(b):
        def row_body(t, carry):
            for k in range(NREG):
                s, o = k // 8, L * (k % 8)
                wbuf[b][t, pl.ds(L * k, L)] = buf3[b][t, s, pl.ds(o, L)]
            lanes = lax.iota(jnp.int32, L)
            vals = buf3[b][t, 7, pl.ds(96, L)]
            rows_i = jnp.full((L,), t, jnp.int32)
            cols_i = jnp.minimum(NREG * L + lanes, VOCAB - 1)
            plsc.store_scatter(
                wbuf[b], [rows_i, cols_i], vals, mask=lanes < VOCAB - NREG * L
            )
            return carry

        lax.fori_loop(0, T, row_body, 0)

    gather(0, 0).start()
    gather(1, 1).start()

    def group(g, carry):
        for b in range(2):
            j = 2 * g + b
            gather(j, b).wait()

            @pl.when(g >= 1)
            def _():
                write(j - 2, b).wait()

            repack(b)
            write(j, b).start()

            @pl.when(g < BPW // 2 - 1)
            def _():
                gather(j + 2, b).start()
        return carry

    lax.fori_loop(0, BPW // 2, group, 0)
    write(BPW - 2, 0).wait()
    write(BPW - 1, 1).wait()


@jax.jit
def _run(idx, table3):
    mesh = plsc.VectorSubcoreMesh(core_axis_name="c", subcore_axis_name="s")
    return pl.kernel(
        _gather_body,
        out_type=jax.ShapeDtypeStruct((B, T, VOCAB), jnp.float32),
        mesh=mesh,
        scratch_types=[
            pltpu.VMEM((BPW, T), jnp.int32),
            pltpu.VMEM((T, 8, 128), jnp.float32),
            pltpu.VMEM((T, 8, 128), jnp.float32),
            pltpu.VMEM((T, VOCAB), jnp.float32),
            pltpu.VMEM((T, VOCAB), jnp.float32),
            pltpu.SemaphoreType.DMA,
            pltpu.SemaphoreType.DMA,
            pltpu.SemaphoreType.DMA,
            pltpu.SemaphoreType.DMA,
        ],
        compiler_params=pltpu.CompilerParams(needs_layout_passes=False),
    )(table3, idx)


def kernel(idx, token_embedding_table):
    table3 = jnp.pad(
        token_embedding_table, ((0, 0), (0, 8 * 128 - VOCAB))
    ).reshape(VOCAB, 8, 128)
    return _run(idx, table3)


# speedup vs baseline: 1.6326x; 1.6326x over previous
import jax
import jax.numpy as jnp
from jax import lax
from jax.experimental import pallas as pl
from jax.experimental.pallas import tpu as pltpu
from jax.experimental.pallas import tpu_sc as plsc

VOCAB = 1000
B = 4096
T = 20
NC = 2
NS = 16
NW = NC * NS
BPW = B // NW
NT = 8
MAIN = 896


def _gather_body(table_hbm, idx_hbm, out_hbm, tail_hbm, idx_v, bufa, bufb,
                 g0, g1, w0, w1):
    buf = (bufa, bufb)
    gsem = (g0, g1)
    wsem = (w0, w1)

    sid = lax.axis_index("s")
    wid = sid * NC + lax.axis_index("c")
    base = wid * BPW
    pltpu.sync_copy(idx_hbm.at[pl.ds(base, BPW)], idx_v)

    def gather(j, b):
        return pltpu.make_async_copy(
            table_hbm.at[idx_v.at[j]], buf[b], gsem[b]
        )

    def write_main(j, b):
        return pltpu.make_async_copy(
            buf[b].at[:, pl.ds(0, MAIN)],
            out_hbm.at[base + j, :, pl.ds(0, MAIN)],
            wsem[b],
        )

    def write_tail(j, b):
        return pltpu.make_async_copy(
            buf[b].at[:, pl.ds(MAIN, 128)], tail_hbm.at[base + j], wsem[b]
        )

    gather(0, 0).start()
    gather(1, 1).start()

    def group(g, carry):
        for b in range(2):
            j = 2 * g + b
            gather(j, b).wait()

            @pl.when(g >= 1)
            def _():
                write_main(j - 2, b).wait()
                write_tail(j - 2, b).wait()

            write_main(j, b).start()
            write_tail(j, b).start()

            @pl.when(g < BPW // 2 - 1)
            def _():
                gather(j + 2, b).start()
        return carry

    lax.fori_loop(0, BPW // 2, group, 0)
    for j, b in ((BPW - 2, 0), (BPW - 1, 1)):
        write_main(j, b).wait()
        write_tail(j, b).wait()


@jax.jit
def _run(idx, table3):
    mesh = plsc.VectorSubcoreMesh(core_axis_name="c", subcore_axis_name="s")
    out, tail = pl.kernel(
        _gather_body,
        out_type=(
            jax.ShapeDtypeStruct((B, T, VOCAB), jnp.float32),
            jax.ShapeDtypeStruct((B, T, 128), jnp.float32),
        ),
        mesh=mesh,
        scratch_types=[
            pltpu.VMEM((BPW, T), jnp.int32),
            pltpu.VMEM((T, NT * 128), jnp.float32),
            pltpu.VMEM((T, NT * 128), jnp.float32),
            pltpu.SemaphoreType.DMA,
            pltpu.SemaphoreType.DMA,
            pltpu.SemaphoreType.DMA,
            pltpu.SemaphoreType.DMA,
        ],
    )(table3, idx)
    return lax.dynamic_update_slice(
        out, lax.slice(tail, (0, 0, 0), (B, T, VOCAB - MAIN)), (0, 0, MAIN)
    )


def kernel(idx, token_embedding_table):
    table_pad = jnp.pad(token_embedding_table, ((0, 0), (0, NT * 128 - VOCAB)))
    return _run(idx, table_pad)


# speedup vs baseline: 1.7062x; 1.0451x over previous
import jax
import jax.numpy as jnp
from jax import lax
from jax.experimental import pallas as pl
from jax.experimental.pallas import tpu as pltpu
from jax.experimental.pallas import tpu_sc as plsc

VOCAB = 1000
B = 4096
T = 20
NC = 2
NS = 16
NW = NC * NS
BPW = B // NW
NT = 8
MAIN = 896


def _gather_body(table_hbm, idx_hbm, out_hbm, tail_hbm, idx_v, bufa, bufb,
                 g0, g1, w0, w1):
    buf = (bufa, bufb)
    gsem = (g0, g1)
    wsem = (w0, w1)

    sid = lax.axis_index("s")
    wid = sid * NC + lax.axis_index("c")
    base = wid * BPW
    pltpu.sync_copy(idx_hbm.at[pl.ds(base, BPW)], idx_v)

    def gather(j, b):
        return pltpu.make_async_copy(
            table_hbm.at[idx_v.at[j]], buf[b], gsem[b]
        )

    def write_main(j, b):
        return pltpu.make_async_copy(
            buf[b].reshape(T, NT * 128).at[:, pl.ds(0, MAIN)],
            out_hbm.at[base + j, :, pl.ds(0, MAIN)],
            wsem[b],
        )

    def write_tail(j, b):
        return pltpu.make_async_copy(
            buf[b].reshape(T, NT * 128).at[:, pl.ds(MAIN, 128)],
            tail_hbm.at[base + j],
            wsem[b],
        )

    gather(0, 0).start()
    gather(1, 1).start()

    def group(g, carry):
        for b in range(2):
            j = 2 * g + b
            gather(j, b).wait()

            @pl.when(g >= 1)
            def _():
                write_main(j - 2, b).wait()
                write_tail(j - 2, b).wait()

            write_main(j, b).start()
            write_tail(j, b).start()

            @pl.when(g < BPW // 2 - 1)
            def _():
                gather(j + 2, b).start()
        return carry

    lax.fori_loop(0, BPW // 2, group, 0)
    for j, b in ((BPW - 2, 0), (BPW - 1, 1)):
        write_main(j, b).wait()
        write_tail(j, b).wait()


@jax.jit
def _run(idx, table3):
    mesh = plsc.VectorSubcoreMesh(core_axis_name="c", subcore_axis_name="s")
    out, tail = pl.kernel(
        _gather_body,
        out_type=(
            jax.ShapeDtypeStruct((B, T, VOCAB), jnp.float32),
            jax.ShapeDtypeStruct((B, T, 128), jnp.float32),
        ),
        mesh=mesh,
        scratch_types=[
            pltpu.VMEM((BPW, T), jnp.int32),
            pltpu.VMEM((T, NT, 128), jnp.float32),
            pltpu.VMEM((T, NT, 128), jnp.float32),
            pltpu.SemaphoreType.DMA,
            pltpu.SemaphoreType.DMA,
            pltpu.SemaphoreType.DMA,
            pltpu.SemaphoreType.DMA,
        ],
    )(table3, idx)
    return lax.dynamic_update_slice(
        out, lax.slice(tail, (0, 0, 0), (B, T, VOCAB - MAIN)), (0, 0, MAIN)
    )


def kernel(idx, token_embedding_table):
    table3 = jnp.pad(
        token_embedding_table, ((0, 0), (0, NT * 128 - VOCAB))
    ).reshape(VOCAB, NT, 128)
    return _run(idx, table3)
